# Initial kernel scaffold; baseline (speedup 1.0000x reference)
import jax
import jax.numpy as jnp
from jax.experimental import pallas as pl


def kernel(avf_out, seq_len, W1, b1, W2, b2, W3, b3):
    raise NotImplementedError("write your pallas kernel here")


# speedup vs baseline: 2.9025x; 2.9025x over previous
import jax
import jax.numpy as jnp
from jax.experimental import pallas as pl
from jax.experimental.pallas import tpu as pltpu

B, T, D = 16, 4096, 128
H1, H2 = 512, 32
TT = 1024


def _mlp_body(x_ref, w1_ref, b1_ref, w2_ref, b2_ref, w3_ref, b3_ref, o_ref):
    x = x_ref[...].astype(jnp.bfloat16)
    w1 = w1_ref[...].astype(jnp.bfloat16)
    h = jax.lax.dot_general(x, w1, (((1,), (0,)), ((), ())),
                            preferred_element_type=jnp.float32)
    h = jnp.maximum(h + b1_ref[...], 0.0).astype(jnp.bfloat16)
    w2 = w2_ref[...].astype(jnp.bfloat16)
    h2 = jax.lax.dot_general(h, w2, (((1,), (0,)), ((), ())),
                             preferred_element_type=jnp.float32)
    h2 = h2 + b2_ref[...]
    h3 = jnp.sum(h2 * w3_ref[...], axis=1) + b3_ref[0, 0]
    o_ref[...] = jax.nn.sigmoid(h3).reshape(TT // 128, 128)


def _select_body(lg_ref, sl_ref, o_ref):
    x = lg_ref[...]
    sl = sl_ref[...].reshape(B, 1)
    k = sl // 16 + 1
    pos = jax.lax.broadcasted_iota(jnp.int32, (B, T), 1)
    xm = jnp.where(pos < sl, x, -1.0)
    t = jnp.zeros((B, 1), jnp.int32)
    for bit in range(30, -1, -1):
        cand = t | (1 << bit)
        cf = jax.lax.bitcast_convert_type(cand, jnp.float32)
        cnt = jnp.sum((xm >= cf).astype(jnp.int32), axis=1, keepdims=True)
        t = jnp.where(cnt >= k, cand, t)
    tf = jax.lax.bitcast_convert_type(t, jnp.float32)
    ex = jnp.sum(jnp.where(xm > tf, xm - tf, 0.0), axis=1, keepdims=True)
    kf = k.astype(jnp.float32)
    o_ref[...] = ((kf * tf + ex) / kf).reshape(1, B)


def kernel(avf_out, seq_len, W1, b1, W2, b2, W3, b3):
    x2d = avf_out.reshape(B * T, D)
    logits = pl.pallas_call(
        _mlp_body,
        grid=(B * T // TT,),
        in_specs=[
            pl.BlockSpec((TT, D), lambda i: (i, 0)),
            pl.BlockSpec((D, H1), lambda i: (0, 0)),
            pl.BlockSpec((1, H1), lambda i: (0, 0)),
            pl.BlockSpec((H1, H2), lambda i: (0, 0)),
            pl.BlockSpec((1, H2), lambda i: (0, 0)),
            pl.BlockSpec((1, H2), lambda i: (0, 0)),
            pl.BlockSpec((1, 1), lambda i: (0, 0)),
        ],
        out_specs=pl.BlockSpec((TT // 128, 128), lambda i: (i, 0)),
        out_shape=jax.ShapeDtypeStruct((B * T // 128, 128), jnp.float32),
    )(x2d, W1, b1.reshape(1, H1), W2, b2.reshape(1, H2),
      W3.reshape(1, H2), b3.reshape(1, 1))

    out = pl.pallas_call(
        _select_body,
        out_shape=jax.ShapeDtypeStruct((1, B), jnp.float32),
    )(logits.reshape(B, T), seq_len.reshape(1, B))
    return out.reshape(B)


# speedup vs baseline: 2.9712x; 1.0237x over previous
import dataclasses
import functools

import jax
import jax.numpy as jnp
from jax.experimental import pallas as pl
from jax.experimental.pallas import tpu as pltpu
from jax.experimental.pallas import tpu_sc as plsc

B, T, D = 16, 4096, 128
H1, H2 = 512, 32
TT = 8192


def _mlp_body(x_ref, w1_ref, b1_ref, w2_ref, b2_ref, w3_ref, b3_ref, o_ref):
    x = x_ref[...].astype(jnp.bfloat16)
    w1 = w1_ref[...].astype(jnp.bfloat16)
    h = jax.lax.dot_general(x, w1, (((1,), (0,)), ((), ())),
                            preferred_element_type=jnp.float32)
    h = jnp.maximum(h + b1_ref[...], 0.0).astype(jnp.bfloat16)
    w2 = w2_ref[...].astype(jnp.bfloat16)
    h2 = jax.lax.dot_general(h, w2, (((1,), (0,)), ((), ())),
                             preferred_element_type=jnp.float32)
    h2 = h2 + b2_ref[...]
    h3 = jnp.sum(h2 * w3_ref[...], axis=1) + b3_ref[0, 0]
    o_ref[...] = jax.nn.sigmoid(h3).reshape(TT // 128, 128)


_LANES = 16


def _sc_select_body(lgf_hbm, sl_hbm, o_hbm, fbuf, svbuf, obuf, sem, sem2):
    c = jax.lax.axis_index("c")
    s = jax.lax.axis_index("s")

    @pl.when(c == 0)
    def _():
        bag = s
        cp_sl = pltpu.async_copy(sl_hbm, svbuf, sem2)
        pltpu.async_copy(lgf_hbm.at[bag], fbuf.at[pl.ds(0, T)], sem).wait()
        cp_sl.wait()

        lane = jax.lax.iota(jnp.int32, _LANES)
        sv = svbuf[...]
        bag_vec = jnp.full((_LANES,), bag, jnp.int32)
        L = jax.lax.reduce_max(jnp.where(lane == bag_vec, sv, 0), (0,))
        k = L // 16 + 1
        nc = (L + _LANES - 1) // _LANES

        neg1f = jnp.full((_LANES,), -1.0, jnp.float32)
        base = (nc - 1) * _LANES
        bm = base + lane < jnp.full((_LANES,), L, jnp.int32)
        fbuf[pl.ds(base, _LANES)] = jnp.where(bm, fbuf[pl.ds(base, _LANES)],
                                              neg1f)
        for g in range(1, 8):
            fbuf[pl.ds((nc - 1 + g) * _LANES, _LANES)] = neg1f

        k_vec = jnp.full((_LANES,), k, jnp.int32)
        one = jnp.full((_LANES,), 1, jnp.int32)
        zero = jnp.full((_LANES,), 0, jnp.int32)

        t = zero
        for sh in range(28, -1, -2):
            c1 = plsc.bitcast(t | jnp.full((_LANES,), 1 << sh, jnp.int32),
                              jnp.float32)
            c2 = plsc.bitcast(t | jnp.full((_LANES,), 2 << sh, jnp.int32),
                              jnp.float32)
            c3 = plsc.bitcast(t | jnp.full((_LANES,), 3 << sh, jnp.int32),
                              jnp.float32)

            def cnt_body(i, accs, c1=c1, c2=c2, c3=c3):
                a1, a2, a3, b1_, b2_, b3_ = accs
                b0 = i * (2 * _LANES)
                x = fbuf[pl.ds(b0, _LANES)]
                y = fbuf[pl.ds(b0 + _LANES, _LANES)]
                return (a1 + jnp.where(x >= c1, one, zero),
                        a2 + jnp.where(x >= c2, one, zero),
                        a3 + jnp.where(x >= c3, one, zero),
                        b1_ + jnp.where(y >= c1, one, zero),
                        b2_ + jnp.where(y >= c2, one, zero),
                        b3_ + jnp.where(y >= c3, one, zero))

            a1, a2, a3, b1_, b2_, b3_ = jax.lax.fori_loop(
                0, (nc + 1) // 2, cnt_body, (zero,) * 6)
            t1 = jnp.full((_LANES,), jax.lax.reduce_sum(a1 + b1_, (0,)),
                          jnp.int32)
            t2 = jnp.full((_LANES,), jax.lax.reduce_sum(a2 + b2_, (0,)),
                          jnp.int32)
            t3 = jnp.full((_LANES,), jax.lax.reduce_sum(a3 + b3_, (0,)),
                          jnp.int32)
            t = jnp.where(t3 >= k_vec, plsc.bitcast(c3, jnp.int32),
                          jnp.where(t2 >= k_vec, plsc.bitcast(c2, jnp.int32),
                                    jnp.where(t1 >= k_vec,
                                              plsc.bitcast(c1, jnp.int32), t)))

        tf = plsc.bitcast(t, jnp.float32)
        fz = jnp.full((_LANES,), 0.0, jnp.float32)

        def ex_body(i, accs):
            b0 = i * (8 * _LANES)
            xs = [fbuf[pl.ds(b0 + u * _LANES, _LANES)] for u in range(8)]
            return tuple(a + jnp.where(x > tf, x - tf, fz)
                         for a, x in zip(accs, xs))

        eaccs = jax.lax.fori_loop(0, (nc + 7) // 8, ex_body, (fz,) * 8)
        ex = jax.lax.reduce_sum(functools.reduce(jnp.add, eaccs), (0,))
        kf_vec = k_vec.astype(jnp.float32)
        ex_vec = jnp.full((_LANES,), ex, jnp.float32)
        obuf[...] = (kf_vec * tf + ex_vec) / kf_vec
        pltpu.async_copy(obuf, o_hbm.at[bag], sem).wait()


def _sc_select(logits, seq_len):
    mesh = plsc.VectorSubcoreMesh(core_axis_name="c", subcore_axis_name="s")
    cp = pltpu.CompilerParams()
    if "needs_layout_passes" in pltpu.CompilerParams.__dataclass_fields__:
        cp = dataclasses.replace(cp, needs_layout_passes=False)
    run = pl.kernel(
        _sc_select_body,
        out_type=jax.ShapeDtypeStruct((B, _LANES), jnp.float32),
        mesh=mesh,
        compiler_params=cp,
        scratch_types=[
            pltpu.VMEM((T + 128,), jnp.float32),
            pltpu.VMEM((_LANES,), jnp.int32),
            pltpu.VMEM((_LANES,), jnp.float32),
            pltpu.SemaphoreType.DMA,
            pltpu.SemaphoreType.DMA,
        ],
    )
    return run(logits, seq_len)[:, 0]


def kernel(avf_out, seq_len, W1, b1, W2, b2, W3, b3):
    x2d = avf_out.reshape(B * T, D)
    logits = pl.pallas_call(
        _mlp_body,
        grid=(B * T // TT,),
        in_specs=[
            pl.BlockSpec((TT, D), lambda i: (i, 0)),
            pl.BlockSpec((D, H1), lambda i: (0, 0)),
            pl.BlockSpec((1, H1), lambda i: (0, 0)),
            pl.BlockSpec((H1, H2), lambda i: (0, 0)),
            pl.BlockSpec((1, H2), lambda i: (0, 0)),
            pl.BlockSpec((1, H2), lambda i: (0, 0)),
            pl.BlockSpec((1, 1), lambda i: (0, 0)),
        ],
        out_specs=pl.BlockSpec((TT // 128, 128), lambda i: (i, 0)),
        out_shape=jax.ShapeDtypeStruct((B * T // 128, 128), jnp.float32),
    )(x2d, W1, b1.reshape(1, H1), W2, b2.reshape(1, H2),
      W3.reshape(1, H2), b3.reshape(1, 1))

    return _sc_select(logits.reshape(B, T), seq_len)
